# baseline (speedup 1.0000x reference)
import functools

import jax
import jax.numpy as jnp
from jax import lax
from jax.experimental import pallas as pl
from jax.experimental.pallas import tpu as pltpu
from jax.experimental.pallas import tpu_sc as plsc

N = 10000
NPAD = 10240
E = 320000
EPAD = 327680
NCH = 160
NCH2 = 80
NCHD = 80
RPT = NPAD // 16
EPS = 1e-5
R = 1000
G = N // R

_mesh = plsc.VectorSubcoreMesh(core_axis_name="c", subcore_axis_name="s")


def _const_fill(buf, nrow, dh, val):
    per_row = dh // 16

    def body(k, _):
        r = k // per_row
        q = k % per_row
        buf[r, pl.ds(q * 16, 16)] = jnp.full((16,), val, jnp.float32)
        return _

    lax.fori_loop(0, nrow * per_row, body, None)


def _make_sc_agg(nq):
    phases = nq // 2

    @functools.partial(
        pl.kernel,
        mesh=_mesh,
        out_type=jax.ShapeDtypeStruct((nq, NPAD, 64), jnp.float32),
        compiler_params=pltpu.CompilerParams(use_tc_tiling_on_sc=False),
        scratch_types=[
            pltpu.VMEM((NCH2, 256), jnp.int32),
            pltpu.VMEM((NCH2, 256), jnp.int32),
            pltpu.VMEM((256, 64), jnp.float32),
            pltpu.VMEM((256, 64), jnp.float32),
            pltpu.VMEM_SHARED((NPAD, 64), jnp.float32),
            pltpu.SemaphoreType.DMA,
            pltpu.SemaphoreType.DMA,
        ],
    )
    def agg(table_hbm, src_hbm, dst_hbm, out_hbm, src_v, dst_v, rows_a, rows_b,
            acc_sh, sem_a, sem_b):
        c = lax.axis_index("c")
        s = lax.axis_index("s")
        pltpu.sync_copy(dst_hbm.at[s], dst_v)

        for phase in range(phases):
            q = c * phases + phase
            _const_fill(rows_a, 256, 64, 0.0)
            for i in range(RPT // 128):
                pltpu.sync_copy(
                    rows_a.at[pl.ds(0, 128)],
                    acc_sh.at[pl.ds(s * RPT + i * 128, 128)])
            plsc.subcore_barrier()

            pltpu.sync_copy(src_hbm.at[q, s], src_v)
            pltpu.async_copy(table_hbm.at[src_v.at[0]], rows_a, sem_a)

            def step2(jj, _):
                j0 = 2 * jj
                j1 = j0 + 1
                j2 = j0 + 2
                pltpu.async_copy(table_hbm.at[src_v.at[j1]], rows_b, sem_b)
                pltpu.make_async_copy(table_hbm.at[src_v.at[j0]], rows_a, sem_a).wait()
                pltpu.sync_copy(rows_a, acc_sh.at[dst_v.at[j0]], add=True)

                @pl.when(j2 < NCH2)
                def _():
                    pltpu.async_copy(table_hbm.at[src_v.at[j2]], rows_a, sem_a)

                pltpu.make_async_copy(table_hbm.at[src_v.at[j1]], rows_b, sem_b).wait()
                pltpu.sync_copy(rows_b, acc_sh.at[dst_v.at[j1]], add=True)
                return _

            lax.fori_loop(0, NCH2 // 2, step2, None)
            plsc.subcore_barrier()

            pltpu.sync_copy(
                acc_sh.at[pl.ds(s * RPT, RPT)], out_hbm.at[q, pl.ds(s * RPT, RPT)]
            )

    return agg


_sc_agg4 = _make_sc_agg(4)
_sc_agg2 = _make_sc_agg(2)


@functools.partial(
    pl.kernel,
    mesh=_mesh,
    out_type=jax.ShapeDtypeStruct((2, NPAD, 16), jnp.float32),
    compiler_params=pltpu.CompilerParams(use_tc_tiling_on_sc=False),
    scratch_types=[
        pltpu.VMEM((NCHD, 128), jnp.int32),
        pltpu.VMEM((128, 16), jnp.float32),
        pltpu.VMEM_SHARED((NPAD, 16), jnp.float32),
    ],
)
def _sc_deg(dst_hbm, out_hbm, dst_v, ones_v, acc_sh):
    c = lax.axis_index("c")
    s = lax.axis_index("s")

    _const_fill(ones_v, 128, 16, 0.0)
    for i in range(RPT // 128):
        pltpu.sync_copy(ones_v, acc_sh.at[pl.ds(s * RPT + i * 128, 128)])
    plsc.subcore_barrier()

    _const_fill(ones_v, 128, 16, 1.0)
    pltpu.sync_copy(dst_hbm.at[c, s], dst_v)

    def step(j, _):
        pltpu.sync_copy(ones_v, acc_sh.at[dst_v.at[j]], add=True)
        return _

    lax.fori_loop(0, NCHD, step, None)
    plsc.subcore_barrier()

    pltpu.sync_copy(acc_sh.at[pl.ds(s * RPT, RPT)], out_hbm.at[c, pl.ds(s * RPT, RPT)])


def _ka_body(x_ref, w_ref, dp_ref, tab_ref, dis_ref):
    deg = dp_ref[0, :, 0:1] + dp_ref[1, :, 0:1] + 1.0
    dis = lax.rsqrt(deg)
    h = jnp.dot(x_ref[...], w_ref[...], preferred_element_type=jnp.float32)
    hs = h * dis
    for q in range(4):
        tab_ref[q] = hs[:, q * 64 : (q + 1) * 64]
    dis_ref[...] = dis


def _tc_first(x, w0, degp):
    return pl.pallas_call(
        _ka_body,
        grid=(G,),
        in_specs=[
            pl.BlockSpec((R, 128), lambda i: (i, 0)),
            pl.BlockSpec((128, 256), lambda i: (0, 0)),
            pl.BlockSpec((2, R, 16), lambda i: (0, i, 0)),
        ],
        out_specs=[
            pl.BlockSpec((4, R, 64), lambda i: (0, i, 0)),
            pl.BlockSpec((R, 1), lambda i: (i, 0)),
        ],
        out_shape=[
            jax.ShapeDtypeStruct((4, N, 64), jnp.float32),
            jax.ShapeDtypeStruct((N, 1), jnp.float32),
        ],
    )(x, w0, degp)


def _layer_body(agg_ref, tab_ref, dis_ref, b_ref, g_ref, be_ref, w_ref,
                tab_o, acc, *, nq, dn):
    p = pl.program_id(0)
    i = pl.program_id(1)
    t = jnp.concatenate([agg_ref[q] + tab_ref[q] for q in range(nq)], axis=1)
    t = t * dis_ref[...] + b_ref[...]

    @pl.when(p == 0)
    def _():
        @pl.when(i == 0)
        def _():
            acc[...] = jnp.zeros_like(acc)

        acc[0:1, :] += jnp.sum(t, axis=0, keepdims=True)
        acc[1:2, :] += jnp.sum(t * t, axis=0, keepdims=True)

    @pl.when(p == 1)
    def _():
        mean = acc[0:1, :] * (1.0 / N)
        var = acc[1:2, :] * (1.0 / N) - mean * mean
        inv = lax.rsqrt(var + EPS)
        y = jnp.maximum((t - mean) * inv * g_ref[...] + be_ref[...], 0.0)
        h = jnp.dot(y, w_ref[...], preferred_element_type=jnp.float32)
        hs = h * dis_ref[...]
        for q in range(dn // 64):
            tab_o[q] = hs[:, q * 64 : (q + 1) * 64]


def _tc_layer(agg, tab, dis, b, g, be, w, d, dn):
    nq = d // 64
    nqo = dn // 64
    return pl.pallas_call(
        functools.partial(_layer_body, nq=nq, dn=dn),
        grid=(2, G),
        in_specs=[
            pl.BlockSpec((nq, R, 64), lambda p, i: (0, i, 0)),
            pl.BlockSpec((nq, R, 64), lambda p, i: (0, i, 0)),
            pl.BlockSpec((R, 1), lambda p, i: (i, 0)),
            pl.BlockSpec((1, d), lambda p, i: (0, 0)),
            pl.BlockSpec((1, d), lambda p, i: (0, 0)),
            pl.BlockSpec((1, d), lambda p, i: (0, 0)),
            pl.BlockSpec((d, dn), lambda p, i: (0, 0)),
        ],
        out_specs=pl.BlockSpec((nqo, R, 64), lambda p, i: (0, i, 0)),
        out_shape=jax.ShapeDtypeStruct((nqo, N, 64), jnp.float32),
        scratch_shapes=[pltpu.VMEM((2, d), jnp.float32)],
    )(agg, tab, dis, b, g, be, w)


def _kc_body(agg_ref, tab_ref, dis_ref, b_ref, o_ref):
    t = jnp.concatenate([agg_ref[q] + tab_ref[q] for q in range(2)], axis=1)
    t = t * dis_ref[...] + b_ref[...]
    m = jnp.max(t, axis=1, keepdims=True)
    lse = jnp.log(jnp.sum(jnp.exp(t - m), axis=1, keepdims=True)) + m
    o_ref[...] = t - lse


def _tc_final(agg, tab, dis, b):
    return pl.pallas_call(
        _kc_body,
        grid=(G,),
        in_specs=[
            pl.BlockSpec((2, R, 64), lambda i: (0, i, 0)),
            pl.BlockSpec((2, R, 64), lambda i: (0, i, 0)),
            pl.BlockSpec((R, 1), lambda i: (i, 0)),
            pl.BlockSpec((1, 128), lambda i: (0, 0)),
        ],
        out_specs=pl.BlockSpec((R, 128), lambda i: (i, 0)),
        out_shape=jax.ShapeDtypeStruct((N, 128), jnp.float32),
    )(agg, tab, dis, b)


@jax.jit
def kernel(x, edge_index, W0, b0, g0, be0, W1, b1, g1, be1, W2, b2):
    src = edge_index[0]
    dst = edge_index[1]
    pad = EPAD - E
    src_p = jnp.concatenate([src, jnp.zeros((pad,), jnp.int32)])
    dst_p = jnp.concatenate([dst, jnp.full((pad,), N, jnp.int32)])
    src_q4 = jnp.stack([src_p + q * N for q in range(4)]).reshape(4, 16, NCH2, 256)
    src_q2 = src_q4[:2].reshape(2, 16, NCH2, 256)
    dst_main = dst_p.reshape(16, NCH2, 256)
    dst_deg = dst_p.reshape(2, 16, NCHD, 128)

    degp = _sc_deg(dst_deg)

    tab0, dis = _tc_first(x, W0, degp)
    agg1 = _sc_agg4(tab0.reshape(4 * N, 64), src_q4, dst_main)
    tab1 = _tc_layer(agg1, tab0, dis, b0.reshape(1, 256),
                     g0.reshape(1, 256), be0.reshape(1, 256), W1, 256, 256)

    agg2 = _sc_agg4(tab1.reshape(4 * N, 64), src_q4, dst_main)
    tab2 = _tc_layer(agg2, tab1, dis, b1.reshape(1, 256),
                     g1.reshape(1, 256), be1.reshape(1, 256), W2, 256, 128)

    agg3 = _sc_agg2(tab2.reshape(2 * N, 64), src_q2, dst_main)
    return _tc_final(agg3, tab2, dis, b2.reshape(1, 128))


# speedup vs baseline: 1.3539x; 1.3539x over previous
import functools

import jax
import jax.numpy as jnp
from jax import lax
from jax.experimental import pallas as pl
from jax.experimental.pallas import tpu as pltpu
from jax.experimental.pallas import tpu_sc as plsc

N = 10000
NPAD = 10240
E = 320000
EPAD = 323584
NCH = 158
NCHD = 79
RPT = NPAD // 16
EPS = 1e-5
R = 1000
G = N // R

_mesh = plsc.VectorSubcoreMesh(core_axis_name="c", subcore_axis_name="s")


def _const_fill(buf, nrow, dh, val):
    per_row = dh // 16

    def body(k, _):
        r = k // per_row
        q = k % per_row
        buf[r, pl.ds(q * 16, 16)] = jnp.full((16,), val, jnp.float32)
        return _

    lax.fori_loop(0, nrow * per_row, body, None)


def _make_sc_agg(nq):
    phases = nq // 2

    @functools.partial(
        pl.kernel,
        mesh=_mesh,
        out_type=jax.ShapeDtypeStruct((nq, NPAD, 64), jnp.float32),
        compiler_params=pltpu.CompilerParams(use_tc_tiling_on_sc=False),
        scratch_types=[
            pltpu.VMEM((NCH, 128), jnp.int32),
            pltpu.VMEM((NCH, 128), jnp.int32),
            pltpu.VMEM((128, 64), jnp.float32),
            pltpu.VMEM((128, 64), jnp.float32),
            pltpu.VMEM_SHARED((NPAD, 64), jnp.float32),
            pltpu.SemaphoreType.DMA,
            pltpu.SemaphoreType.DMA,
        ],
    )
    def agg(table_hbm, src_hbm, dst_hbm, out_hbm, src_v, dst_v, rows_a, rows_b,
            acc_sh, sem_a, sem_b):
        c = lax.axis_index("c")
        s = lax.axis_index("s")
        pltpu.sync_copy(dst_hbm.at[s], dst_v)

        for phase in range(phases):
            q = c * phases + phase
            _const_fill(rows_a, 128, 64, 0.0)
            for i in range(RPT // 128):
                pltpu.sync_copy(rows_a, acc_sh.at[pl.ds(s * RPT + i * 128, 128)])
            plsc.subcore_barrier()

            pltpu.sync_copy(src_hbm.at[q, s], src_v)
            pltpu.async_copy(table_hbm.at[src_v.at[0]], rows_a, sem_a)

            def step2(jj, _):
                j0 = 2 * jj
                j1 = j0 + 1
                j2 = j0 + 2
                pltpu.async_copy(table_hbm.at[src_v.at[j1]], rows_b, sem_b)
                pltpu.make_async_copy(table_hbm.at[src_v.at[j0]], rows_a, sem_a).wait()
                pltpu.sync_copy(rows_a, acc_sh.at[dst_v.at[j0]], add=True)

                @pl.when(j2 < NCH)
                def _():
                    pltpu.async_copy(table_hbm.at[src_v.at[j2]], rows_a, sem_a)

                pltpu.make_async_copy(table_hbm.at[src_v.at[j1]], rows_b, sem_b).wait()
                pltpu.sync_copy(rows_b, acc_sh.at[dst_v.at[j1]], add=True)
                return _

            lax.fori_loop(0, NCH // 2, step2, None)
            plsc.subcore_barrier()

            pltpu.sync_copy(
                acc_sh.at[pl.ds(s * RPT, RPT)], out_hbm.at[q, pl.ds(s * RPT, RPT)]
            )

    return agg


_sc_agg4 = _make_sc_agg(4)
_sc_agg2 = _make_sc_agg(2)


@functools.partial(
    pl.kernel,
    mesh=_mesh,
    out_type=jax.ShapeDtypeStruct((2, NPAD, 16), jnp.float32),
    compiler_params=pltpu.CompilerParams(use_tc_tiling_on_sc=False),
    scratch_types=[
        pltpu.VMEM((NCHD, 128), jnp.int32),
        pltpu.VMEM((128, 16), jnp.float32),
        pltpu.VMEM_SHARED((NPAD, 16), jnp.float32),
    ],
)
def _sc_deg(dst_hbm, out_hbm, dst_v, ones_v, acc_sh):
    c = lax.axis_index("c")
    s = lax.axis_index("s")

    _const_fill(ones_v, 128, 16, 0.0)
    for i in range(RPT // 128):
        pltpu.sync_copy(ones_v, acc_sh.at[pl.ds(s * RPT + i * 128, 128)])
    plsc.subcore_barrier()

    _const_fill(ones_v, 128, 16, 1.0)
    pltpu.sync_copy(dst_hbm.at[c, s], dst_v)

    def step(j, _):
        pltpu.sync_copy(ones_v, acc_sh.at[dst_v.at[j]], add=True)
        return _

    lax.fori_loop(0, NCHD, step, None)
    plsc.subcore_barrier()

    pltpu.sync_copy(acc_sh.at[pl.ds(s * RPT, RPT)], out_hbm.at[c, pl.ds(s * RPT, RPT)])


def _mm0_body(x_ref, w_ref, h_ref):
    h_ref[...] = jnp.dot(x_ref[...], w_ref[...],
                         preferred_element_type=jnp.float32)


def _tc_matmul0(x, w0):
    return pl.pallas_call(
        _mm0_body,
        grid=(G,),
        in_specs=[
            pl.BlockSpec((R, 128), lambda i: (i, 0)),
            pl.BlockSpec((128, 256), lambda i: (0, 0)),
        ],
        out_specs=pl.BlockSpec((R, 256), lambda i: (i, 0)),
        out_shape=jax.ShapeDtypeStruct((N, 256), jnp.float32),
    )(x, w0)


def _ka_body(h_ref, dp_ref, tab_ref, dis_ref):
    deg = dp_ref[0, :, 0:1] + dp_ref[1, :, 0:1] + 1.0
    dis = lax.rsqrt(deg)
    hs = h_ref[...] * dis
    for q in range(4):
        tab_ref[q] = hs[:, q * 64 : (q + 1) * 64]
    dis_ref[...] = dis


def _tc_scale0(h, degp):
    return pl.pallas_call(
        _ka_body,
        grid=(G,),
        in_specs=[
            pl.BlockSpec((R, 256), lambda i: (i, 0)),
            pl.BlockSpec((2, R, 16), lambda i: (0, i, 0)),
        ],
        out_specs=[
            pl.BlockSpec((4, R, 64), lambda i: (0, i, 0)),
            pl.BlockSpec((R, 1), lambda i: (i, 0)),
        ],
        out_shape=[
            jax.ShapeDtypeStruct((4, N, 64), jnp.float32),
            jax.ShapeDtypeStruct((N, 1), jnp.float32),
        ],
    )(h, degp)


def _layer_body(agg_ref, tab_ref, dis_ref, b_ref, g_ref, be_ref, w_ref,
                tab_o, acc, *, nq, dn):
    p = pl.program_id(0)
    i = pl.program_id(1)
    t = jnp.concatenate([agg_ref[q] + tab_ref[q] for q in range(nq)], axis=1)
    t = t * dis_ref[...] + b_ref[...]

    @pl.when(p == 0)
    def _():
        @pl.when(i == 0)
        def _():
            acc[...] = jnp.zeros_like(acc)

        acc[0:1, :] += jnp.sum(t, axis=0, keepdims=True)
        acc[1:2, :] += jnp.sum(t * t, axis=0, keepdims=True)

    @pl.when(p == 1)
    def _():
        mean = acc[0:1, :] * (1.0 / N)
        var = acc[1:2, :] * (1.0 / N) - mean * mean
        inv = lax.rsqrt(var + EPS)
        y = jnp.maximum((t - mean) * inv * g_ref[...] + be_ref[...], 0.0)
        h = jnp.dot(y, w_ref[...], preferred_element_type=jnp.float32)
        hs = h * dis_ref[...]
        for q in range(dn // 64):
            tab_o[q] = hs[:, q * 64 : (q + 1) * 64]


def _tc_layer(agg, tab, dis, b, g, be, w, d, dn):
    nq = d // 64
    nqo = dn // 64
    return pl.pallas_call(
        functools.partial(_layer_body, nq=nq, dn=dn),
        grid=(2, G),
        in_specs=[
            pl.BlockSpec((nq, R, 64), lambda p, i: (0, i, 0)),
            pl.BlockSpec((nq, R, 64), lambda p, i: (0, i, 0)),
            pl.BlockSpec((R, 1), lambda p, i: (i, 0)),
            pl.BlockSpec((1, d), lambda p, i: (0, 0)),
            pl.BlockSpec((1, d), lambda p, i: (0, 0)),
            pl.BlockSpec((1, d), lambda p, i: (0, 0)),
            pl.BlockSpec((d, dn), lambda p, i: (0, 0)),
        ],
        out_specs=pl.BlockSpec((nqo, R, 64), lambda p, i: (0, i, 0)),
        out_shape=jax.ShapeDtypeStruct((nqo, N, 64), jnp.float32),
        scratch_shapes=[pltpu.VMEM((2, d), jnp.float32)],
    )(agg, tab, dis, b, g, be, w)


def _kc_body(agg_ref, tab_ref, dis_ref, b_ref, o_ref):
    t = jnp.concatenate([agg_ref[q] + tab_ref[q] for q in range(2)], axis=1)
    t = t * dis_ref[...] + b_ref[...]
    m = jnp.max(t, axis=1, keepdims=True)
    lse = jnp.log(jnp.sum(jnp.exp(t - m), axis=1, keepdims=True)) + m
    o_ref[...] = t - lse


def _tc_final(agg, tab, dis, b):
    return pl.pallas_call(
        _kc_body,
        grid=(G,),
        in_specs=[
            pl.BlockSpec((2, R, 64), lambda i: (0, i, 0)),
            pl.BlockSpec((2, R, 64), lambda i: (0, i, 0)),
            pl.BlockSpec((R, 1), lambda i: (i, 0)),
            pl.BlockSpec((1, 128), lambda i: (0, 0)),
        ],
        out_specs=pl.BlockSpec((R, 128), lambda i: (i, 0)),
        out_shape=jax.ShapeDtypeStruct((N, 128), jnp.float32),
    )(agg, tab, dis, b)


@jax.jit
def kernel(x, edge_index, W0, b0, g0, be0, W1, b1, g1, be1, W2, b2):
    src = edge_index[0]
    dst = edge_index[1]
    pad = EPAD - E
    src_p = jnp.concatenate([src, jnp.zeros((pad,), jnp.int32)])
    dst_p = jnp.concatenate([dst, jnp.full((pad,), N, jnp.int32)])
    src_q4 = jnp.stack([src_p + q * N for q in range(4)]).reshape(4, 16, NCH, 128)
    src_q2 = src_q4[:2].reshape(2, 16, NCH, 128)
    dst_main = dst_p.reshape(16, NCH, 128)
    dst_deg = dst_p.reshape(2, 16, NCHD, 128)

    h0 = _tc_matmul0(x, W0)
    degp = _sc_deg(dst_deg)
    tab0, dis = _tc_scale0(h0, degp)
    agg1 = _sc_agg4(tab0.reshape(4 * N, 64), src_q4, dst_main)
    tab1 = _tc_layer(agg1, tab0, dis, b0.reshape(1, 256),
                     g0.reshape(1, 256), be0.reshape(1, 256), W1, 256, 256)

    agg2 = _sc_agg4(tab1.reshape(4 * N, 64), src_q4, dst_main)
    tab2 = _tc_layer(agg2, tab1, dis, b1.reshape(1, 256),
                     g1.reshape(1, 256), be1.reshape(1, 256), W2, 256, 128)

    agg3 = _sc_agg2(tab2.reshape(2 * N, 64), src_q2, dst_main)
    return _tc_final(agg3, tab2, dis, b2.reshape(1, 128))


# speedup vs baseline: 1.3812x; 1.0201x over previous
import functools

import jax
import jax.numpy as jnp
from jax import lax
from jax.experimental import pallas as pl
from jax.experimental.pallas import tpu as pltpu
from jax.experimental.pallas import tpu_sc as plsc

N = 10000
NPAD = 10240
E = 320000
EPAD = 323584
NCH = 158
NCHD = 79
RPT = NPAD // 16
EPS = 1e-5
R = 1000
G = N // R

_mesh = plsc.VectorSubcoreMesh(core_axis_name="c", subcore_axis_name="s")


def _const_fill(buf, nrow, dh, val):
    per_row = dh // 16

    def body(k, _):
        r = k // per_row
        q = k % per_row
        buf[r, pl.ds(q * 16, 16)] = jnp.full((16,), val, jnp.float32)
        return _

    lax.fori_loop(0, nrow * per_row, body, None)


def _make_sc_agg(nq):
    phases = nq // 2

    @functools.partial(
        pl.kernel,
        mesh=_mesh,
        out_type=jax.ShapeDtypeStruct((nq, NPAD, 64), jnp.float32),
        compiler_params=pltpu.CompilerParams(use_tc_tiling_on_sc=False),
        scratch_types=[
            pltpu.VMEM((NCH, 128), jnp.int32),
            pltpu.VMEM((NCH, 128), jnp.int32),
            pltpu.VMEM((128, 64), jnp.float32),
            pltpu.VMEM((128, 64), jnp.float32),
            pltpu.VMEM((128, 64), jnp.float32),
            pltpu.VMEM_SHARED((NPAD, 64), jnp.float32),
            pltpu.SemaphoreType.DMA,
            pltpu.SemaphoreType.DMA,
        ],
    )
    def agg(table_hbm, src_hbm, dst_hbm, out_hbm, src_v, dst_v, rows_a, rows_b,
            zbuf, acc_sh, sem_a, sem_b):
        c = lax.axis_index("c")
        s = lax.axis_index("s")

        def pipeline_loop():
            def step2(jj, _):
                j0 = 2 * jj
                j1 = j0 + 1
                j2 = j0 + 2
                pltpu.async_copy(table_hbm.at[src_v.at[j1]], rows_b, sem_b)
                pltpu.make_async_copy(table_hbm.at[src_v.at[j0]], rows_a, sem_a).wait()
                pltpu.sync_copy(rows_a, acc_sh.at[dst_v.at[j0]], add=True)

                @pl.when(j2 < NCH)
                def _():
                    pltpu.async_copy(table_hbm.at[src_v.at[j2]], rows_a, sem_a)

                pltpu.make_async_copy(table_hbm.at[src_v.at[j1]], rows_b, sem_b).wait()
                pltpu.sync_copy(rows_b, acc_sh.at[dst_v.at[j1]], add=True)
                return _

            lax.fori_loop(0, NCH // 2, step2, None)
            plsc.subcore_barrier()

        def zero_own_rows():
            for i in range(RPT // 128):
                pltpu.sync_copy(zbuf, acc_sh.at[pl.ds(s * RPT + i * 128, 128)])

        pltpu.sync_copy(dst_hbm.at[s], dst_v)
        pltpu.sync_copy(src_hbm.at[c * phases, s], src_v)
        pltpu.async_copy(table_hbm.at[src_v.at[0]], rows_a, sem_a)
        _const_fill(zbuf, 128, 64, 0.0)
        zero_own_rows()
        plsc.subcore_barrier()
        pipeline_loop()

        for phase in range(1, phases):
            q = c * phases + phase
            pltpu.sync_copy(src_hbm.at[q, s], src_v)
            pltpu.async_copy(table_hbm.at[src_v.at[0]], rows_a, sem_a)
            pltpu.sync_copy(
                acc_sh.at[pl.ds(s * RPT, RPT)],
                out_hbm.at[q - 1, pl.ds(s * RPT, RPT)],
            )
            zero_own_rows()
            plsc.subcore_barrier()
            pipeline_loop()

        pltpu.sync_copy(
            acc_sh.at[pl.ds(s * RPT, RPT)],
            out_hbm.at[c * phases + phases - 1, pl.ds(s * RPT, RPT)],
        )

    return agg


_sc_agg4 = _make_sc_agg(4)
_sc_agg2 = _make_sc_agg(2)


@functools.partial(
    pl.kernel,
    mesh=_mesh,
    out_type=jax.ShapeDtypeStruct((2, NPAD, 16), jnp.float32),
    compiler_params=pltpu.CompilerParams(use_tc_tiling_on_sc=False),
    scratch_types=[
        pltpu.VMEM((NCHD, 128), jnp.int32),
        pltpu.VMEM((128, 16), jnp.float32),
        pltpu.VMEM_SHARED((NPAD, 16), jnp.float32),
    ],
)
def _sc_deg(dst_hbm, out_hbm, dst_v, ones_v, acc_sh):
    c = lax.axis_index("c")
    s = lax.axis_index("s")

    _const_fill(ones_v, 128, 16, 0.0)
    for i in range(RPT // 128):
        pltpu.sync_copy(ones_v, acc_sh.at[pl.ds(s * RPT + i * 128, 128)])
    plsc.subcore_barrier()

    _const_fill(ones_v, 128, 16, 1.0)
    pltpu.sync_copy(dst_hbm.at[c, s], dst_v)

    def step(j, _):
        pltpu.sync_copy(ones_v, acc_sh.at[dst_v.at[j]], add=True)
        return _

    lax.fori_loop(0, NCHD, step, None)
    plsc.subcore_barrier()

    pltpu.sync_copy(acc_sh.at[pl.ds(s * RPT, RPT)], out_hbm.at[c, pl.ds(s * RPT, RPT)])


def _ka_body(x_ref, w_ref, dp_ref, tab_ref, dis_ref):
    deg = dp_ref[0, :, 0:1] + dp_ref[1, :, 0:1] + 1.0
    dis = lax.rsqrt(deg)
    h = jnp.dot(x_ref[...], w_ref[...], preferred_element_type=jnp.float32)
    hs = h * dis
    for q in range(4):
        tab_ref[q] = hs[:, q * 64 : (q + 1) * 64]
    dis_ref[...] = dis


def _tc_first(x, w0, degp):
    return pl.pallas_call(
        _ka_body,
        grid=(G,),
        in_specs=[
            pl.BlockSpec((R, 128), lambda i: (i, 0)),
            pl.BlockSpec((128, 256), lambda i: (0, 0)),
            pl.BlockSpec((2, R, 16), lambda i: (0, i, 0)),
        ],
        out_specs=[
            pl.BlockSpec((4, R, 64), lambda i: (0, i, 0)),
            pl.BlockSpec((R, 1), lambda i: (i, 0)),
        ],
        out_shape=[
            jax.ShapeDtypeStruct((4, N, 64), jnp.float32),
            jax.ShapeDtypeStruct((N, 1), jnp.float32),
        ],
    )(x, w0, degp)


def _layer_body(agg_ref, tab_ref, dis_ref, b_ref, g_ref, be_ref, w_ref,
                tab_o, acc, *, nq, dn):
    p = pl.program_id(0)
    i = pl.program_id(1)
    t = jnp.concatenate([agg_ref[q] + tab_ref[q] for q in range(nq)], axis=1)
    t = t * dis_ref[...] + b_ref[...]

    @pl.when(p == 0)
    def _():
        @pl.when(i == 0)
        def _():
            acc[...] = jnp.zeros_like(acc)

        acc[0:1, :] += jnp.sum(t, axis=0, keepdims=True)
        acc[1:2, :] += jnp.sum(t * t, axis=0, keepdims=True)

    @pl.when(p == 1)
    def _():
        mean = acc[0:1, :] * (1.0 / N)
        var = acc[1:2, :] * (1.0 / N) - mean * mean
        inv = lax.rsqrt(var + EPS)
        y = jnp.maximum((t - mean) * inv * g_ref[...] + be_ref[...], 0.0)
        h = jnp.dot(y, w_ref[...], preferred_element_type=jnp.float32)
        hs = h * dis_ref[...]
        for q in range(dn // 64):
            tab_o[q] = hs[:, q * 64 : (q + 1) * 64]


def _tc_layer(agg, tab, dis, b, g, be, w, d, dn):
    nq = d // 64
    nqo = dn // 64
    return pl.pallas_call(
        functools.partial(_layer_body, nq=nq, dn=dn),
        grid=(2, G),
        in_specs=[
            pl.BlockSpec((nq, R, 64), lambda p, i: (0, i, 0)),
            pl.BlockSpec((nq, R, 64), lambda p, i: (0, i, 0)),
            pl.BlockSpec((R, 1), lambda p, i: (i, 0)),
            pl.BlockSpec((1, d), lambda p, i: (0, 0)),
            pl.BlockSpec((1, d), lambda p, i: (0, 0)),
            pl.BlockSpec((1, d), lambda p, i: (0, 0)),
            pl.BlockSpec((d, dn), lambda p, i: (0, 0)),
        ],
        out_specs=pl.BlockSpec((nqo, R, 64), lambda p, i: (0, i, 0)),
        out_shape=jax.ShapeDtypeStruct((nqo, N, 64), jnp.float32),
        scratch_shapes=[pltpu.VMEM((2, d), jnp.float32)],
    )(agg, tab, dis, b, g, be, w)


def _kc_body(agg_ref, tab_ref, dis_ref, b_ref, o_ref):
    t = jnp.concatenate([agg_ref[q] + tab_ref[q] for q in range(2)], axis=1)
    t = t * dis_ref[...] + b_ref[...]
    m = jnp.max(t, axis=1, keepdims=True)
    lse = jnp.log(jnp.sum(jnp.exp(t - m), axis=1, keepdims=True)) + m
    o_ref[...] = t - lse


def _tc_final(agg, tab, dis, b):
    return pl.pallas_call(
        _kc_body,
        grid=(G,),
        in_specs=[
            pl.BlockSpec((2, R, 64), lambda i: (0, i, 0)),
            pl.BlockSpec((2, R, 64), lambda i: (0, i, 0)),
            pl.BlockSpec((R, 1), lambda i: (i, 0)),
            pl.BlockSpec((1, 128), lambda i: (0, 0)),
        ],
        out_specs=pl.BlockSpec((R, 128), lambda i: (i, 0)),
        out_shape=jax.ShapeDtypeStruct((N, 128), jnp.float32),
    )(agg, tab, dis, b)


@jax.jit
def kernel(x, edge_index, W0, b0, g0, be0, W1, b1, g1, be1, W2, b2):
    src = edge_index[0]
    dst = edge_index[1]
    pad = EPAD - E
    src_p = jnp.concatenate([src, jnp.zeros((pad,), jnp.int32)])
    dst_p = jnp.concatenate([dst, jnp.full((pad,), N, jnp.int32)])
    src_q4 = jnp.stack([src_p + q * N for q in range(4)]).reshape(4, 16, NCH, 128)
    src_q2 = src_q4[:2].reshape(2, 16, NCH, 128)
    dst_main = dst_p.reshape(16, NCH, 128)
    dst_deg = dst_p.reshape(2, 16, NCHD, 128)

    degp = _sc_deg(dst_deg)

    tab0, dis = _tc_first(x, W0, degp)
    agg1 = _sc_agg4(tab0.reshape(4 * N, 64), src_q4, dst_main)
    tab1 = _tc_layer(agg1, tab0, dis, b0.reshape(1, 256),
                     g0.reshape(1, 256), be0.reshape(1, 256), W1, 256, 256)

    agg2 = _sc_agg4(tab1.reshape(4 * N, 64), src_q4, dst_main)
    tab2 = _tc_layer(agg2, tab1, dis, b1.reshape(1, 256),
                     g1.reshape(1, 256), be1.reshape(1, 256), W2, 256, 128)

    agg3 = _sc_agg2(tab2.reshape(2 * N, 64), src_q2, dst_main)
    return _tc_final(agg3, tab2, dis, b2.reshape(1, 128))
